# initial kernel scaffold (unmeasured)
import jax
import jax.numpy as jnp
from jax import lax
from jax.experimental import pallas as pl
from jax.experimental.pallas import tpu as pltpu

N_DEV = 16


def kernel(x, w_mat, scale_x, scale_w):
    m_per, k = x.shape
    _, n_per = w_mat.shape

    def body(x_ref, w_ref, sx_ref, sw_ref, out_ref,
             gather_ref, send_sems, recv_sems):
        my = lax.axis_index("i")
        left = lax.rem(my - 1 + N_DEV, N_DEV)
        right = lax.rem(my + 1, N_DEV)

        barrier_sem = pltpu.get_barrier_semaphore()
        for nbr in (left, right):
            pl.semaphore_signal(
                barrier_sem, inc=1,
                device_id=(nbr,), device_id_type=pl.DeviceIdType.MESH,
            )
        pl.semaphore_wait(barrier_sem, 2)

        scale = sx_ref[0] * sw_ref[0]

        def compute(h):
            o = lax.rem(my - h + N_DEV, N_DEV)
            acc = jnp.dot(gather_ref[h], w_ref[...],
                          preferred_element_type=jnp.float32)
            out_ref[pl.ds(o * m_per, m_per), :] = acc * scale

        gather_ref[0] = x_ref[...]

        for h in range(N_DEV - 1):
            rdma = pltpu.make_async_remote_copy(
                src_ref=gather_ref.at[h],
                dst_ref=gather_ref.at[h + 1],
                send_sem=send_sems.at[h],
                recv_sem=recv_sems.at[h],
                device_id=(right,),
                device_id_type=pl.DeviceIdType.MESH,
            )
            rdma.start()
            compute(h)
            rdma.wait()

        compute(N_DEV - 1)

    return pl.pallas_call(
        body,
        out_shape=jax.ShapeDtypeStruct((N_DEV * m_per, n_per), jnp.float32),
        in_specs=[
            pl.BlockSpec(memory_space=pltpu.VMEM),
            pl.BlockSpec(memory_space=pltpu.VMEM),
            pl.BlockSpec(memory_space=pltpu.SMEM),
            pl.BlockSpec(memory_space=pltpu.SMEM),
        ],
        out_specs=pl.BlockSpec(memory_space=pltpu.VMEM),
        scratch_shapes=[
            pltpu.VMEM((N_DEV, m_per, k), x.dtype),
            pltpu.SemaphoreType.DMA((N_DEV - 1,)),
            pltpu.SemaphoreType.DMA((N_DEV - 1,)),
        ],
        compiler_params=pltpu.CompilerParams(collective_id=0),
    )(x, w_mat, scale_x, scale_w)


# baseline (device time: 209784 ns/iter reference)
import jax
import jax.numpy as jnp
from jax import lax
from jax.experimental import pallas as pl
from jax.experimental.pallas import tpu as pltpu

N_DEV = 16


def kernel(x, w_mat, scale_x, scale_w):
    m_per, k = x.shape
    _, n_per = w_mat.shape

    def body(x_ref, w_ref, sx_ref, sw_ref, out_ref,
             gather_ref, wq_ref, send_sems, recv_sems):
        my = lax.axis_index("i")
        left = lax.rem(my - 1 + N_DEV, N_DEV)
        right = lax.rem(my + 1, N_DEV)

        barrier_sem = pltpu.get_barrier_semaphore()
        for nbr in (left, right):
            pl.semaphore_signal(
                barrier_sem, inc=1,
                device_id=(nbr,), device_id_type=pl.DeviceIdType.MESH,
            )
        pl.semaphore_wait(barrier_sem, 2)

        scale = sx_ref[0] * sw_ref[0]

        def compute(h):
            o = lax.rem(my - h + N_DEV, N_DEV)
            acc = jnp.dot(gather_ref[h], wq_ref[...],
                          preferred_element_type=jnp.float32)
            out_ref[pl.ds(o * m_per, m_per), :] = acc * scale

        gather_ref[0] = x_ref[...].astype(jnp.float8_e5m2)
        wq_ref[...] = w_ref[...].astype(jnp.float8_e5m2)

        for h in range(N_DEV - 1):
            rdma = pltpu.make_async_remote_copy(
                src_ref=gather_ref.at[h],
                dst_ref=gather_ref.at[h + 1],
                send_sem=send_sems.at[h],
                recv_sem=recv_sems.at[h],
                device_id=(right,),
                device_id_type=pl.DeviceIdType.MESH,
            )
            rdma.start()
            compute(h)
            rdma.wait()

        compute(N_DEV - 1)

    return pl.pallas_call(
        body,
        out_shape=jax.ShapeDtypeStruct((N_DEV * m_per, n_per), jnp.float32),
        in_specs=[
            pl.BlockSpec(memory_space=pltpu.VMEM),
            pl.BlockSpec(memory_space=pltpu.VMEM),
            pl.BlockSpec(memory_space=pltpu.SMEM),
            pl.BlockSpec(memory_space=pltpu.SMEM),
        ],
        out_specs=pl.BlockSpec(memory_space=pltpu.VMEM),
        scratch_shapes=[
            pltpu.VMEM((N_DEV, m_per, k), jnp.float8_e5m2),
            pltpu.VMEM((k, n_per), jnp.float8_e5m2),
            pltpu.SemaphoreType.DMA((N_DEV - 1,)),
            pltpu.SemaphoreType.DMA((N_DEV - 1,)),
        ],
        compiler_params=pltpu.CompilerParams(collective_id=0),
    )(x, w_mat, scale_x, scale_w)


# device time: 131164 ns/iter; 1.5994x vs baseline; 1.5994x over previous
import jax
import jax.numpy as jnp
from jax import lax
from jax.experimental import pallas as pl
from jax.experimental.pallas import tpu as pltpu

N_DEV = 16
R_HOPS = N_DEV // 2
L_HOPS = N_DEV // 2 - 1


def kernel(x, w_mat, scale_x, scale_w):
    m_per, k = x.shape
    _, n_per = w_mat.shape

    def body(x_ref, w_ref, sx_ref, sw_ref, out_ref,
             r_buf, l_buf, wq_ref,
             r_send, r_recv, l_send, l_recv):
        my = lax.axis_index("i")
        left = lax.rem(my - 1 + N_DEV, N_DEV)
        right = lax.rem(my + 1, N_DEV)

        barrier_sem = pltpu.get_barrier_semaphore()
        for nbr in (left, right):
            pl.semaphore_signal(
                barrier_sem, inc=1,
                device_id=(nbr,), device_id_type=pl.DeviceIdType.MESH,
            )
        pl.semaphore_wait(barrier_sem, 2)

        scale = sx_ref[0] * sw_ref[0]

        def compute(buf, h, o):
            acc = jnp.dot(buf[h], wq_ref[...],
                          preferred_element_type=jnp.float32)
            out_ref[pl.ds(o * m_per, m_per), :] = acc * scale

        xq = x_ref[...].astype(jnp.float8_e5m2)
        r_buf[0] = xq
        l_buf[0] = xq
        wq_ref[...] = w_ref[...].astype(jnp.float8_e5m2)

        for h in range(R_HOPS):
            r = pltpu.make_async_remote_copy(
                src_ref=r_buf.at[h], dst_ref=r_buf.at[h + 1],
                send_sem=r_send.at[h], recv_sem=r_recv.at[h],
                device_id=(right,), device_id_type=pl.DeviceIdType.MESH,
            )
            r.start()
            if h < L_HOPS:
                l = pltpu.make_async_remote_copy(
                    src_ref=l_buf.at[h], dst_ref=l_buf.at[h + 1],
                    send_sem=l_send.at[h], recv_sem=l_recv.at[h],
                    device_id=(left,), device_id_type=pl.DeviceIdType.MESH,
                )
                l.start()
            compute(r_buf, h, lax.rem(my - h + N_DEV, N_DEV))
            if 1 <= h:
                compute(l_buf, h, lax.rem(my + h, N_DEV))
            r.wait()
            if h < L_HOPS:
                l.wait()

        compute(r_buf, R_HOPS, lax.rem(my - R_HOPS + N_DEV, N_DEV))
        compute(l_buf, L_HOPS, lax.rem(my + L_HOPS, N_DEV))

    return pl.pallas_call(
        body,
        out_shape=jax.ShapeDtypeStruct((N_DEV * m_per, n_per), jnp.float32),
        in_specs=[
            pl.BlockSpec(memory_space=pltpu.VMEM),
            pl.BlockSpec(memory_space=pltpu.VMEM),
            pl.BlockSpec(memory_space=pltpu.SMEM),
            pl.BlockSpec(memory_space=pltpu.SMEM),
        ],
        out_specs=pl.BlockSpec(memory_space=pltpu.VMEM),
        scratch_shapes=[
            pltpu.VMEM((R_HOPS + 1, m_per, k), jnp.float8_e5m2),
            pltpu.VMEM((L_HOPS + 1, m_per, k), jnp.float8_e5m2),
            pltpu.VMEM((k, n_per), jnp.float8_e5m2),
            pltpu.SemaphoreType.DMA((R_HOPS,)),
            pltpu.SemaphoreType.DMA((R_HOPS,)),
            pltpu.SemaphoreType.DMA((L_HOPS,)),
            pltpu.SemaphoreType.DMA((L_HOPS,)),
        ],
        compiler_params=pltpu.CompilerParams(collective_id=0),
    )(x, w_mat, scale_x, scale_w)


# device time: 103741 ns/iter; 2.0222x vs baseline; 1.2643x over previous
import jax
import jax.numpy as jnp
from jax import lax
from jax.experimental import pallas as pl
from jax.experimental.pallas import tpu as pltpu

N_DEV = 16
HOPS = N_DEV // 2
SUB = 2


def _subs(h):
    if h < HOPS - 1:
        return (0, 1)
    return None


def kernel(x, w_mat, scale_x, scale_w):
    m_per, k = x.shape
    _, n_per = w_mat.shape
    sub_m = m_per // SUB

    def body(x_ref, w_ref, sx_ref, sw_ref, out_ref,
             r_buf, l_buf, wq_ref,
             r_send, r_recv, l_send, l_recv):
        my = lax.axis_index("i")
        left = lax.rem(my - 1 + N_DEV, N_DEV)
        right = lax.rem(my + 1, N_DEV)

        barrier_sem = pltpu.get_barrier_semaphore()
        for nbr in (left, right):
            pl.semaphore_signal(
                barrier_sem, inc=1,
                device_id=(nbr,), device_id_type=pl.DeviceIdType.MESH,
            )
        pl.semaphore_wait(barrier_sem, 2)

        scale = sx_ref[0] * sw_ref[0]

        def rdma(buf, sems_pair, h, s, dev):
            send_sems, recv_sems = sems_pair
            return pltpu.make_async_remote_copy(
                src_ref=buf.at[h, s], dst_ref=buf.at[h + 1, s],
                send_sem=send_sems.at[h, s], recv_sem=recv_sems.at[h, s],
                device_id=(dev,), device_id_type=pl.DeviceIdType.MESH,
            )

        def compute(buf, h, o, s):
            acc = jnp.dot(buf[h, s], wq_ref[...],
                          preferred_element_type=jnp.float32)
            out_ref[pl.ds(o * m_per + s * sub_m, sub_m), :] = acc * scale

        xq = x_ref[...].astype(jnp.float8_e5m2)
        r_buf[0, 0] = xq[:sub_m]
        r_buf[0, 1] = xq[sub_m:]
        l_buf[0, 0] = xq[:sub_m]
        l_buf[0, 1] = xq[sub_m:]
        wq_ref[...] = w_ref[...].astype(jnp.float8_e5m2)

        started = []

        def start(buf, sems, h, s, dev):
            d = rdma(buf, sems, h, s, dev)
            d.start()
            started.append(d)
            return d

        r_sems = (r_send, r_recv)
        l_sems = (l_send, l_recv)
        inflight = {}
        for s in (0, 1):
            inflight[("r", 0, s)] = start(r_buf, r_sems, 0, s, right)
            inflight[("l", 0, s)] = start(l_buf, l_sems, 0, s, left)
        compute(r_buf, 0, my, 0)
        compute(r_buf, 0, my, 1)

        for h in range(1, HOPS):
            subs_r = (0, 1) if h < HOPS - 1 else (0,)
            subs_l = (0, 1) if h < HOPS - 1 else (1,)
            for s in (0, 1):
                inflight[("r", h - 1, s)].wait_recv()
                if s in subs_r:
                    inflight[("r", h, s)] = start(r_buf, r_sems, h, s, right)
                inflight[("l", h - 1, s)].wait_recv()
                if s in subs_l:
                    inflight[("l", h, s)] = start(l_buf, l_sems, h, s, left)
            o_r = lax.rem(my - h + N_DEV, N_DEV)
            o_l = lax.rem(my + h, N_DEV)
            for s in (0, 1):
                compute(r_buf, h, o_r, s)
                compute(l_buf, h, o_l, s)

        o8 = lax.rem(my + HOPS, N_DEV)
        inflight[("r", HOPS - 1, 0)].wait_recv()
        compute(r_buf, HOPS, o8, 0)
        inflight[("l", HOPS - 1, 1)].wait_recv()
        compute(l_buf, HOPS, o8, 1)
        for d in started:
            d.wait_send()

    return pl.pallas_call(
        body,
        out_shape=jax.ShapeDtypeStruct((N_DEV * m_per, n_per), jnp.float32),
        in_specs=[
            pl.BlockSpec(memory_space=pltpu.VMEM),
            pl.BlockSpec(memory_space=pltpu.VMEM),
            pl.BlockSpec(memory_space=pltpu.SMEM),
            pl.BlockSpec(memory_space=pltpu.SMEM),
        ],
        out_specs=pl.BlockSpec(memory_space=pltpu.VMEM),
        scratch_shapes=[
            pltpu.VMEM((HOPS + 1, SUB, sub_m, k), jnp.float8_e5m2),
            pltpu.VMEM((HOPS + 1, SUB, sub_m, k), jnp.float8_e5m2),
            pltpu.VMEM((k, n_per), jnp.float8_e5m2),
            pltpu.SemaphoreType.DMA((HOPS, SUB)),
            pltpu.SemaphoreType.DMA((HOPS, SUB)),
            pltpu.SemaphoreType.DMA((HOPS, SUB)),
            pltpu.SemaphoreType.DMA((HOPS, SUB)),
        ],
        compiler_params=pltpu.CompilerParams(collective_id=0),
    )(x, w_mat, scale_x, scale_w)


# device time: 103177 ns/iter; 2.0332x vs baseline; 1.0055x over previous
import jax
import jax.numpy as jnp
from jax import lax
from jax.experimental import pallas as pl
from jax.experimental.pallas import tpu as pltpu

N_DEV = 16
HOPS = N_DEV // 2
SUB = 4
ALL_SUBS = tuple(range(SUB))
R_LAST = tuple(range(SUB // 2))
L_LAST = tuple(range(SUB // 2, SUB))


def kernel(x, w_mat, scale_x, scale_w):
    m_per, k = x.shape
    _, n_per = w_mat.shape
    sub_m = m_per // SUB

    def body(x_ref, w_ref, sx_ref, sw_ref, out_ref,
             r_buf, l_buf, wq_ref,
             r_send, r_recv, l_send, l_recv):
        my = lax.axis_index("i")
        left = lax.rem(my - 1 + N_DEV, N_DEV)
        right = lax.rem(my + 1, N_DEV)

        barrier_sem = pltpu.get_barrier_semaphore()
        for nbr in (left, right):
            pl.semaphore_signal(
                barrier_sem, inc=1,
                device_id=(nbr,), device_id_type=pl.DeviceIdType.MESH,
            )
        pl.semaphore_wait(barrier_sem, 2)

        started = []
        inflight = {}

        def start(ring, buf, send_sems, recv_sems, h, s, dev):
            d = pltpu.make_async_remote_copy(
                src_ref=buf.at[h, s], dst_ref=buf.at[h + 1, s],
                send_sem=send_sems.at[h, s], recv_sem=recv_sems.at[h, s],
                device_id=(dev,), device_id_type=pl.DeviceIdType.MESH,
            )
            d.start()
            started.append(d)
            inflight[(ring, h, s)] = d

        def compute(buf, h, o, s):
            acc = jnp.dot(buf[h, s], wq_ref[...],
                          preferred_element_type=jnp.float32)
            out_ref[pl.ds(o * m_per + s * sub_m, sub_m), :] = acc * scale

        for s in ALL_SUBS:
            xs = x_ref[pl.ds(s * sub_m, sub_m), :].astype(jnp.float8_e5m2)
            r_buf[0, s] = xs
            l_buf[0, s] = xs
            start("r", r_buf, r_send, r_recv, 0, s, right)
            start("l", l_buf, l_send, l_recv, 0, s, left)

        scale = sx_ref[0] * sw_ref[0]
        wq_ref[...] = w_ref[...].astype(jnp.float8_e5m2)
        for s in ALL_SUBS:
            compute(r_buf, 0, my, s)

        for h in range(1, HOPS):
            subs_r = ALL_SUBS if h < HOPS - 1 else R_LAST
            subs_l = ALL_SUBS if h < HOPS - 1 else L_LAST
            for s in ALL_SUBS:
                inflight[("r", h - 1, s)].wait_recv()
                if s in subs_r:
                    start("r", r_buf, r_send, r_recv, h, s, right)
                inflight[("l", h - 1, s)].wait_recv()
                if s in subs_l:
                    start("l", l_buf, l_send, l_recv, h, s, left)
            o_r = lax.rem(my - h + N_DEV, N_DEV)
            o_l = lax.rem(my + h, N_DEV)
            for s in ALL_SUBS:
                compute(r_buf, h, o_r, s)
                compute(l_buf, h, o_l, s)

        o8 = lax.rem(my + HOPS, N_DEV)
        for s in R_LAST:
            inflight[("r", HOPS - 1, s)].wait_recv()
            compute(r_buf, HOPS, o8, s)
        for s in L_LAST:
            inflight[("l", HOPS - 1, s)].wait_recv()
            compute(l_buf, HOPS, o8, s)
        for d in started:
            d.wait_send()

    return pl.pallas_call(
        body,
        out_shape=jax.ShapeDtypeStruct((N_DEV * m_per, n_per), jnp.float32),
        in_specs=[
            pl.BlockSpec(memory_space=pltpu.VMEM),
            pl.BlockSpec(memory_space=pltpu.VMEM),
            pl.BlockSpec(memory_space=pltpu.SMEM),
            pl.BlockSpec(memory_space=pltpu.SMEM),
        ],
        out_specs=pl.BlockSpec(memory_space=pltpu.VMEM),
        scratch_shapes=[
            pltpu.VMEM((HOPS + 1, SUB, sub_m, k), jnp.float8_e5m2),
            pltpu.VMEM((HOPS + 1, SUB, sub_m, k), jnp.float8_e5m2),
            pltpu.VMEM((k, n_per), jnp.float8_e5m2),
            pltpu.SemaphoreType.DMA((HOPS, SUB)),
            pltpu.SemaphoreType.DMA((HOPS, SUB)),
            pltpu.SemaphoreType.DMA((HOPS, SUB)),
            pltpu.SemaphoreType.DMA((HOPS, SUB)),
        ],
        compiler_params=pltpu.CompilerParams(collective_id=0),
    )(x, w_mat, scale_x, scale_w)


# device time: 102186 ns/iter; 2.0530x vs baseline; 1.0097x over previous
import jax
import jax.numpy as jnp
from jax import lax
from jax.experimental import pallas as pl
from jax.experimental.pallas import tpu as pltpu

N_DEV = 16
HOPS = N_DEV // 2
SUB = 4
ALL_SUBS = tuple(range(SUB))
R_LAST = tuple(range(SUB // 2))
L_LAST = tuple(range(SUB // 2, SUB))

_HAM = (0, 1, 5, 9, 13, 14, 10, 6, 2, 3, 7, 11, 15, 12, 8, 4)
_POS = tuple(_HAM.index(p) for p in range(N_DEV))


def kernel(x, w_mat, scale_x, scale_w):
    m_per, k = x.shape
    _, n_per = w_mat.shape
    sub_m = m_per // SUB

    def body(x_ref, w_ref, sx_ref, sw_ref, ham_ref, pos_ref, out_ref,
             r_buf, l_buf, wq_ref,
             r_send, r_recv, l_send, l_recv):
        my = lax.axis_index("i")
        p = pos_ref[my]
        left = ham_ref[lax.rem(p - 1 + N_DEV, N_DEV)]
        right = ham_ref[lax.rem(p + 1, N_DEV)]

        barrier_sem = pltpu.get_barrier_semaphore()
        for nbr in (left, right):
            pl.semaphore_signal(
                barrier_sem, inc=1,
                device_id=(nbr,), device_id_type=pl.DeviceIdType.MESH,
            )
        pl.semaphore_wait(barrier_sem, 2)

        started = []
        inflight = {}

        def start(ring, buf, send_sems, recv_sems, h, s, dev):
            d = pltpu.make_async_remote_copy(
                src_ref=buf.at[h, s], dst_ref=buf.at[h + 1, s],
                send_sem=send_sems.at[h, s], recv_sem=recv_sems.at[h, s],
                device_id=(dev,), device_id_type=pl.DeviceIdType.MESH,
            )
            d.start()
            started.append(d)
            inflight[(ring, h, s)] = d

        def compute(buf, h, o, s):
            acc = jnp.dot(buf[h, s], wq_ref[...],
                          preferred_element_type=jnp.float32)
            out_ref[pl.ds(o * m_per + s * sub_m, sub_m), :] = acc * scale

        for s in ALL_SUBS:
            xs = x_ref[pl.ds(s * sub_m, sub_m), :].astype(jnp.float8_e5m2)
            r_buf[0, s] = xs
            l_buf[0, s] = xs
            start("r", r_buf, r_send, r_recv, 0, s, right)
            start("l", l_buf, l_send, l_recv, 0, s, left)

        scale = sx_ref[0] * sw_ref[0]
        wq_ref[...] = w_ref[...].astype(jnp.float8_e5m2)
        for s in ALL_SUBS:
            compute(r_buf, 0, my, s)

        for h in range(1, HOPS):
            subs_r = ALL_SUBS if h < HOPS - 1 else R_LAST
            subs_l = ALL_SUBS if h < HOPS - 1 else L_LAST
            for s in ALL_SUBS:
                inflight[("r", h - 1, s)].wait_recv()
                if s in subs_r:
                    start("r", r_buf, r_send, r_recv, h, s, right)
                inflight[("l", h - 1, s)].wait_recv()
                if s in subs_l:
                    start("l", l_buf, l_send, l_recv, h, s, left)
            o_r = ham_ref[lax.rem(p - h + N_DEV, N_DEV)]
            o_l = ham_ref[lax.rem(p + h, N_DEV)]
            for s in ALL_SUBS:
                compute(r_buf, h, o_r, s)
                compute(l_buf, h, o_l, s)

        o8 = ham_ref[lax.rem(p + HOPS, N_DEV)]
        for s in R_LAST:
            inflight[("r", HOPS - 1, s)].wait_recv()
            compute(r_buf, HOPS, o8, s)
        for s in L_LAST:
            inflight[("l", HOPS - 1, s)].wait_recv()
            compute(l_buf, HOPS, o8, s)
        for d in started:
            d.wait_send()

    return pl.pallas_call(
        body,
        out_shape=jax.ShapeDtypeStruct((N_DEV * m_per, n_per), jnp.float32),
        in_specs=[
            pl.BlockSpec(memory_space=pltpu.VMEM),
            pl.BlockSpec(memory_space=pltpu.VMEM),
            pl.BlockSpec(memory_space=pltpu.SMEM),
            pl.BlockSpec(memory_space=pltpu.SMEM),
            pl.BlockSpec(memory_space=pltpu.SMEM),
            pl.BlockSpec(memory_space=pltpu.SMEM),
        ],
        out_specs=pl.BlockSpec(memory_space=pltpu.VMEM),
        scratch_shapes=[
            pltpu.VMEM((HOPS + 1, SUB, sub_m, k), jnp.float8_e5m2),
            pltpu.VMEM((HOPS + 1, SUB, sub_m, k), jnp.float8_e5m2),
            pltpu.VMEM((k, n_per), jnp.float8_e5m2),
            pltpu.SemaphoreType.DMA((HOPS, SUB)),
            pltpu.SemaphoreType.DMA((HOPS, SUB)),
            pltpu.SemaphoreType.DMA((HOPS, SUB)),
            pltpu.SemaphoreType.DMA((HOPS, SUB)),
        ],
        compiler_params=pltpu.CompilerParams(collective_id=0),
    )(x, w_mat, scale_x, scale_w,
      jnp.array(_HAM, dtype=jnp.int32), jnp.array(_POS, dtype=jnp.int32))
